# Initial kernel scaffold; baseline (speedup 1.0000x reference)
import jax
import jax.numpy as jnp
from jax.experimental import pallas as pl


def kernel(x, proj_W, proj_b, proc_W, proc_b, bn_gamma, bn_beta, comp_W, comp_b, unzip_W, unzip_b, unproc_W, unproc_b, rest_W, rest_b):
    raise NotImplementedError("write your pallas kernel here")


# speedup vs baseline: 4.4873x; 4.4873x over previous
import functools

import jax
import jax.numpy as jnp
from jax import lax
from jax.experimental import pallas as pl
from jax.experimental.pallas import tpu as pltpu
from jax.experimental.pallas import tpu_sc as plsc

F32 = jnp.float32
I32 = jnp.int32
U32 = jnp.uint32


def _excl_cumsum_lanes(v):
    n = v.shape[1]
    incl = v
    s = 1
    while s < n:
        shifted = jnp.concatenate(
            [jnp.zeros((1, s), F32), incl[:, : n - s]], axis=1)
        incl = incl + shifted
        s *= 2
    return incl - v


def _k1_body(x_ref, w_ref, pw_ref, pb_ref, tk_ref, src_ref, m_ref, h_ref):
    b = pl.program_id(0)
    T = x_ref.shape[1]
    R = x_ref.shape[2]
    K = 256

    xb = x_ref[0]
    ss = jnp.sum(xb * xb, axis=0, keepdims=True)
    inv = 1.0 / jnp.maximum(jnp.sqrt(ss), 1e-12)
    xn = xb * inv

    w = w_ref[...]
    score = lax.dot_general(w, xn, (((1,), (0,)), ((), ())),
                            preferred_element_type=F32)

    bits = lax.bitcast_convert_type(score, U32)
    neg = (bits >> jnp.uint32(31)) == jnp.uint32(1)
    u = jnp.where(neg, ~bits, bits | jnp.uint32(0x80000000))

    def srch(i, t):
        t2 = t | (jnp.uint32(1) << (jnp.uint32(31) - i.astype(U32)))
        cnt = jnp.sum((u >= t2).astype(I32))
        return jnp.where(cnt >= K, t2, t)

    v = lax.fori_loop(0, 32, srch, jnp.uint32(0))

    gt = u > v
    eq = u == v
    n_gt = jnp.sum(gt.astype(I32))
    tie_budget = (K - n_gt).astype(F32)
    eq_pos = _excl_cumsum_lanes(eq.astype(F32))
    sel = gt | (eq & (eq_pos < tie_budget))
    mrow = sel.astype(F32)
    m_ref[0] = mrow

    ps = _excl_cumsum_lanes(mrow)

    iota_k_col = lax.broadcasted_iota(I32, (K, R), 0).astype(F32)
    A = jnp.where((ps == iota_k_col) & sel, 1.0, 0.0)

    uh = (u >> jnp.uint32(16)).astype(F32)
    ul = (u & jnp.uint32(0xFFFF)).astype(F32)
    ri = lax.broadcasted_iota(I32, (1, R), 1).astype(F32)

    def compact_col(vec):
        return lax.dot_general(A, vec, (((1,), (1,)), ((), ())),
                               preferred_element_type=F32)

    h_c, l_c, i_c = compact_col(uh), compact_col(ul), compact_col(ri)
    eye = (lax.broadcasted_iota(I32, (K, K), 0)
           == lax.broadcasted_iota(I32, (K, K), 1)).astype(F32)
    h_r = jnp.sum(eye * h_c, axis=0, keepdims=True)
    l_r = jnp.sum(eye * l_c, axis=0, keepdims=True)
    i_r = jnp.sum(eye * i_c, axis=0, keepdims=True)

    heq = h_r == h_c
    vgt = (h_r > h_c) | (heq & (l_r > l_c))
    veq = heq & (l_r == l_c)
    cmp = vgt | (veq & (i_r < i_c))
    rank = jnp.sum(cmp.astype(F32), axis=1, keepdims=True)

    iota_k_row = lax.broadcasted_iota(I32, (K, K), 1).astype(F32)
    r1h = jnp.where(rank == iota_k_row, 1.0, 0.0)
    tk = jnp.sum(r1h * i_c, axis=0, keepdims=True)
    tk_ref[0] = tk.astype(I32)

    rank_full = jnp.sum(A * rank, axis=0, keepdims=True)

    src = jnp.where(sel, rank_full, jnp.float32(K) + ri - ps)
    src_ref[0] = src.astype(I32) + b * R

    iota_kr = lax.broadcasted_iota(I32, (K, R), 0).astype(F32)
    G = jnp.where((rank_full == iota_kr) & sel, 1.0, 0.0)
    x_topk = lax.dot_general(xn, G, (((1,), (1,)), ((), ())),
                             preferred_element_type=F32)
    h = lax.dot_general(x_topk, pw_ref[...], (((1,), (1,)), ((), ())),
                        preferred_element_type=F32) + pb_ref[...]
    h_ref[0] = h


def _k2_body(h_ref, g_ref, bta_ref, cw_ref, cb_ref, zw_ref, zb_ref,
             qw_ref, qb_ref, xc_ref, dt_ref):
    B = h_ref.shape[0]
    h = h_ref[...]
    denom = h.shape[0] * h.shape[2]
    s1 = jnp.sum(h, axis=2, keepdims=True)
    mu = jnp.sum(s1, axis=0, keepdims=True) / denom
    d0 = h - mu
    v1 = jnp.sum(d0 * d0, axis=2, keepdims=True)
    var = jnp.sum(v1, axis=0, keepdims=True) / denom
    rs = lax.rsqrt(var + 1e-5)
    hn = d0 * rs * g_ref[...][None] + bta_ref[...][None]

    cw = cw_ref[...]
    zw = zw_ref[...]
    qw = qw_ref[...]
    zb = zb_ref[...]
    qb = qb_ref[...]
    for b in range(B):
        hb = hn[b]
        xc = lax.dot_general(hb, cw, (((1,), (1,)), ((), ())),
                             preferred_element_type=F32) + cb_ref[...]
        xc_ref[b] = xc
        t1 = lax.dot_general(zw, xc, (((1,), (1,)), ((), ())),
                             preferred_element_type=F32) + zb
        dt = lax.dot_general(qw, t1, (((1,), (0,)), ((), ())),
                             preferred_element_type=F32) + qb
        dt_ref[b] = dt


def _k3_body(dt_ref, rw_ref, rb_ref, e_ref):
    K = dt_ref.shape[1]
    dt = dt_ref[0]
    e_ref[0, :K, :] = dt
    xr = lax.dot_general(rw_ref[...], dt, (((1,), (0,)), ((), ())),
                         preferred_element_type=F32) + rb_ref[...]
    e_ref[0, K:, :] = xr


def _sc_gather(table, idx):
    N, D = table.shape
    info = plsc.get_sparse_core_info()
    nw = info.num_cores * info.num_subcores
    chunk = 128
    per_w = N // nw
    steps = per_w // chunk
    mesh = plsc.VectorSubcoreMesh(core_axis_name="c", subcore_axis_name="s")

    @functools.partial(
        pl.kernel, mesh=mesh,
        out_type=jax.ShapeDtypeStruct((N, D), F32),
        compiler_params=pltpu.CompilerParams(use_tc_tiling_on_sc=False),
        scratch_types=[
            pltpu.VMEM((chunk,), I32),
            pltpu.VMEM((chunk, D), F32),
            pltpu.SemaphoreType.DMA,
        ],
    )
    def k(table_hbm, idx_hbm, out_hbm, idx_v, rows_v, sem):
        wid = lax.axis_index("s") * info.num_cores + lax.axis_index("c")
        base = wid * per_w

        def step(i, carry):
            off = base + i * chunk
            pltpu.sync_copy(idx_hbm.at[pl.ds(off, chunk)], idx_v)
            pltpu.async_copy(table_hbm.at[idx_v], rows_v, sem).wait()
            pltpu.sync_copy(rows_v, out_hbm.at[pl.ds(off, chunk)])
            return carry

        lax.fori_loop(0, steps, step, 0)

    return k(table, idx)


def kernel(x, proj_W, proj_b, proc_W, proc_b, bn_gamma, bn_beta,
           comp_W, comp_b, unzip_W, unzip_b, unproc_W, unproc_b,
           rest_W, rest_b):
    B, T, R = x.shape
    K = proc_W.shape[0]
    K2 = comp_W.shape[0]
    REST = R - K

    tk3, src3, m3, h = pl.pallas_call(
        _k1_body,
        grid=(B,),
        in_specs=[
            pl.BlockSpec((1, T, R), lambda b: (b, 0, 0)),
            pl.BlockSpec((1, T), lambda b: (0, 0)),
            pl.BlockSpec((K, K), lambda b: (0, 0)),
            pl.BlockSpec((1, K), lambda b: (0, 0)),
        ],
        out_specs=[
            pl.BlockSpec((1, 1, K), lambda b: (b, 0, 0)),
            pl.BlockSpec((1, 1, R), lambda b: (b, 0, 0)),
            pl.BlockSpec((1, 1, R), lambda b: (b, 0, 0)),
            pl.BlockSpec((1, T, K), lambda b: (b, 0, 0)),
        ],
        out_shape=[
            jax.ShapeDtypeStruct((B, 1, K), I32),
            jax.ShapeDtypeStruct((B, 1, R), I32),
            jax.ShapeDtypeStruct((B, 1, R), F32),
            jax.ShapeDtypeStruct((B, T, K), F32),
        ],
    )(x, proj_W, proc_W, proc_b.reshape(1, K))

    x_comp, d_T = pl.pallas_call(
        _k2_body,
        out_shape=[
            jax.ShapeDtypeStruct((B, T, K2), F32),
            jax.ShapeDtypeStruct((B, K, T), F32),
        ],
    )(h, jnp.broadcast_to(bn_gamma[:, None], (T, K)),
      jnp.broadcast_to(bn_beta[:, None], (T, K)),
      comp_W, comp_b.reshape(1, K2),
      unzip_W, unzip_b.reshape(K, 1),
      unproc_W, unproc_b.reshape(K, 1))

    e = pl.pallas_call(
        _k3_body,
        grid=(B,),
        in_specs=[
            pl.BlockSpec((1, K, T), lambda b: (b, 0, 0)),
            pl.BlockSpec((REST, K), lambda b: (0, 0)),
            pl.BlockSpec((REST, 1), lambda b: (0, 0)),
        ],
        out_specs=pl.BlockSpec((1, R, T), lambda b: (b, 0, 0)),
        out_shape=jax.ShapeDtypeStruct((B, R, T), F32),
    )(d_T, rest_W, rest_b.reshape(REST, 1))

    rec = _sc_gather(e.reshape(B * R, T), src3.reshape(B * R))
    x_recon = jnp.swapaxes(rec.reshape(B, R, T), 1, 2)

    mask = jnp.broadcast_to(m3.reshape(B, 1, R), (B, T, R))
    border_mask = jnp.zeros((B, T, R), jnp.bool_)
    topk_index = tk3.reshape(B, K)
    return x_recon, x_comp, mask, border_mask, topk_index
